# Initial kernel scaffold; baseline (speedup 1.0000x reference)
import jax
import jax.numpy as jnp
from jax.experimental import pallas as pl


def kernel(pos, vel, charges, params, edge_index, batch):
    raise NotImplementedError("write your pallas kernel here")


# speedup vs baseline: 1.9971x; 1.9971x over previous
import functools

import jax
import jax.numpy as jnp
from jax import lax
from jax.experimental import pallas as pl
from jax.experimental.pallas import tpu as pltpu
from jax.experimental.pallas import tpu_sc as plsc

N = 50000
E = 200000
G = 10000
H = 64

NC, NS = 2, 16
CH = 128

S_NODE = 25088
NP = 2 * S_NODE
ACC_N = 25600
EP = 200704
K_E = EP // NS // CH
K_G = 2 * EP // (NC * NS) // CH
S_G = 5120
GP = 2 * S_G
ACC_G = 5248
NSP = 51200
K_B = NSP // NS // CH

BE = 1024
BN = 1024

@functools.cache
def _sc_mesh():
  return plsc.VectorSubcoreMesh(
      core_axis_name="c", subcore_axis_name="s", num_cores=NC, num_subcores=NS)


def _sc_gather(table, idx4, d, k):
  rows_pt = k * CH

  def body(table_h, idx_h, out_h, idx_v, buf_a, buf_b, sem_a, sem_b):
    cid = lax.axis_index("c")
    sid = lax.axis_index("s")
    base = (cid * NS + sid) * rows_pt
    pltpu.sync_copy(idx_h.at[cid, sid], idx_v)

    def step(j, carry):
      @pl.when(lax.rem(j, 2) == 0)
      def _():
        pltpu.async_copy(table_h.at[idx_v.at[j]], buf_a, sem_a).wait()
        pltpu.sync_copy(buf_a, out_h.at[pl.ds(base + j * CH, CH)])

      @pl.when(lax.rem(j, 2) == 1)
      def _():
        pltpu.async_copy(table_h.at[idx_v.at[j]], buf_b, sem_b).wait()
        pltpu.sync_copy(buf_b, out_h.at[pl.ds(base + j * CH, CH)])
      return carry

    lax.fori_loop(0, k, step, 0, unroll=False)

  f = pl.kernel(
      body,
      out_type=jax.ShapeDtypeStruct((NC * NS * rows_pt, d), jnp.float32),
      mesh=_sc_mesh(),
      compiler_params=pltpu.CompilerParams(use_tc_tiling_on_sc=False),
      scratch_types=[
          pltpu.VMEM((k, CH), jnp.int32),
          pltpu.VMEM((CH, d), jnp.float32),
          pltpu.VMEM((CH, d), jnp.float32),
          pltpu.SemaphoreType.DMA,
          pltpu.SemaphoreType.DMA,
      ],
  )
  return f(table, idx4)


def _sc_scatter_add(vals, idx4, zeros, d, k, s_half, acc_rows):
  z_pt = acc_rows // NS
  o_pt = s_half // NS
  nfull, rem = divmod(z_pt, CH)

  def body(vals_h, idx_h, zeros_h, out_h, idx_v, vbuf, acc):
    cid = lax.axis_index("c")
    sid = lax.axis_index("s")
    pltpu.sync_copy(idx_h.at[cid, sid], idx_v)

    zb = sid * z_pt
    for t in range(nfull):
      pltpu.sync_copy(zeros_h, acc.at[pl.ds(zb + t * CH, CH)])
    if rem:
      pltpu.sync_copy(zeros_h.at[pl.ds(0, rem)],
                      acc.at[pl.ds(zb + nfull * CH, rem)])
    plsc.subcore_barrier()

    tb = sid * (k * CH)

    def step(j, carry):
      pltpu.sync_copy(vals_h.at[pl.ds(tb + j * CH, CH)], vbuf)
      pltpu.sync_copy(vbuf, acc.at[idx_v.at[j]], add=True)
      return carry

    lax.fori_loop(0, k, step, 0, unroll=False)
    plsc.subcore_barrier()

    ob = sid * o_pt
    pltpu.sync_copy(acc.at[pl.ds(ob, o_pt)],
                    out_h.at[pl.ds(cid * s_half + ob, o_pt)])

  f = pl.kernel(
      body,
      out_type=jax.ShapeDtypeStruct((2 * s_half, d), jnp.float32),
      mesh=_sc_mesh(),
      compiler_params=pltpu.CompilerParams(use_tc_tiling_on_sc=False),
      scratch_types=[
          pltpu.VMEM((k, CH), jnp.int32),
          pltpu.VMEM((CH, d), jnp.float32),
          pltpu.VMEM_SHARED((acc_rows, d), jnp.float32),
      ],
  )
  return f(vals, idx4, zeros)


def _silu(x):
  return x * jax.nn.sigmoid(x)


def _sh4(r):
  n2 = jnp.sum(r * r, axis=1, keepdims=True)
  unit = r / jnp.clip(jnp.sqrt(n2), 1e-8, None)
  y0 = jnp.full((r.shape[0], 1), 0.28209479177387814, dtype=r.dtype)
  return jnp.concatenate([y0, 0.4886025119029199 * unit], axis=1)


def _tc_preproc(gpre):
  nb = EP // BE

  def kfn(gd_ref, gs_ref, ea_ref, es_ref):
    gd = gd_ref[...]
    gs = gs_ref[...]
    rel = gs[:, 0:3] - gd[:, 0:3]
    n2 = jnp.sum(rel * rel, axis=1, keepdims=True)
    dist = jnp.sqrt(n2 + 1e-12)
    ea4 = _sh4(rel)
    pc = gs[:, 3:4] * gd[:, 3:4]
    one = jnp.ones((BE, 1), jnp.float32)
    zero = jnp.zeros((BE, 1), jnp.float32)
    ea_ref[...] = jnp.concatenate([ea4, one] + [zero] * 11, axis=1)
    es_ref[...] = jnp.concatenate([ea4, dist, pc, zero, zero], axis=1)

  return pl.pallas_call(
      kfn,
      grid=(nb,),
      in_specs=[
          pl.BlockSpec((BE, 16), lambda i: (i, 0)),
          pl.BlockSpec((BE, 16), lambda i: (i + nb, 0)),
      ],
      out_specs=[
          pl.BlockSpec((BE, 16), lambda i: (i, 0)),
          pl.BlockSpec((BE, 8), lambda i: (i, 0)),
      ],
      out_shape=[
          jax.ShapeDtypeStruct((EP, 16), jnp.float32),
          jax.ShapeDtypeStruct((EP, 8), jnp.float32),
      ],
  )(gpre, gpre)


def _tc_embed(nf, mp, na, w, b):
  def kfn(nf_ref, mp_ref, na_ref, w_ref, b_ref, x0_ref, nat_ref):
    nf = nf_ref[...]
    pos = nf[:, 0:3]
    vel = nf[:, 5:8]
    v2 = jnp.sum(vel * vel, axis=1, keepdims=True)
    vel_abs = jnp.sqrt(v2 + 1e-12)
    vel_emb = _sh4(vel)
    na_v = na_ref[...]
    nattr = na_v[:, 0:4] / jnp.clip(na_v[:, 4:5], 1.0, None) + vel_emb
    mp_v = mp_ref[...]
    mean = mp_v[:, 0:3] / jnp.clip(mp_v[:, 4:5], 1.0, None)
    feat = jnp.concatenate(
        [pos - mean, vel, vel_abs, jnp.zeros((BN, 1), jnp.float32)], axis=1)
    acc = jnp.zeros((BN, H), jnp.float32) + b_ref[...]
    for a in range(4):
      acc = acc + nattr[:, a:a + 1] * jnp.dot(
          feat, w_ref[a], preferred_element_type=jnp.float32)
    x0_ref[...] = acc
    nat_ref[...] = nattr

  nb = NP // BN
  return pl.pallas_call(
      kfn,
      grid=(nb,),
      in_specs=[
          pl.BlockSpec((BN, 16), lambda i: (i, 0)),
          pl.BlockSpec((BN, 16), lambda i: (i, 0)),
          pl.BlockSpec((BN, 16), lambda i: (i, 0)),
          pl.BlockSpec((4, 8, H), lambda i: (0, 0, 0)),
          pl.BlockSpec((1, H), lambda i: (0, 0)),
      ],
      out_specs=[
          pl.BlockSpec((BN, H), lambda i: (i, 0)),
          pl.BlockSpec((BN, 4), lambda i: (i, 0)),
      ],
      out_shape=[
          jax.ShapeDtypeStruct((NP, H), jnp.float32),
          jax.ShapeDtypeStruct((NP, 4), jnp.float32),
      ],
  )(nf, mp, na, w, b)


def _tc_edge(g, escal, wxi, wxj, wd, wp, b1, w2, b2):
  nb = EP // BE

  def kfn(xi_ref, xj_ref, es_ref, wxi_ref, wxj_ref, wd_ref, wp_ref,
          b1_ref, w2_ref, b2_ref, out_ref):
    xi = xi_ref[...]
    xj = xj_ref[...]
    es = es_ref[...]
    d = es[:, 4:5]
    p = es[:, 5:6]
    acc = jnp.zeros((BE, H), jnp.float32) + b1_ref[...]
    for a in range(4):
      t = jnp.dot(xi, wxi_ref[a], preferred_element_type=jnp.float32)
      t = t + jnp.dot(xj, wxj_ref[a], preferred_element_type=jnp.float32)
      t = t + d * wd_ref[a:a + 1, :] + p * wp_ref[a:a + 1, :]
      acc = acc + es[:, a:a + 1] * t
    m1 = _silu(acc)
    acc2 = jnp.zeros((BE, H), jnp.float32) + b2_ref[...]
    for a in range(4):
      acc2 = acc2 + es[:, a:a + 1] * jnp.dot(
          m1, w2_ref[a], preferred_element_type=jnp.float32)
    out_ref[...] = _silu(acc2)

  wspec = pl.BlockSpec((4, H, H), lambda i: (0, 0, 0))
  return pl.pallas_call(
      kfn,
      grid=(nb,),
      in_specs=[
          pl.BlockSpec((BE, H), lambda i: (i, 0)),
          pl.BlockSpec((BE, H), lambda i: (i + nb, 0)),
          pl.BlockSpec((BE, 8), lambda i: (i, 0)),
          wspec,
          wspec,
          pl.BlockSpec((4, H), lambda i: (0, 0)),
          pl.BlockSpec((4, H), lambda i: (0, 0)),
          pl.BlockSpec((1, H), lambda i: (0, 0)),
          wspec,
          pl.BlockSpec((1, H), lambda i: (0, 0)),
      ],
      out_specs=pl.BlockSpec((BE, H), lambda i: (i, 0)),
      out_shape=jax.ShapeDtypeStruct((EP, H), jnp.float32),
  )(g, g, escal, wxi, wxj, wd, wp, b1, w2, b2)


def _tc_node(x, agg, nat, wux, wua, b1, wu2, b2):
  def kfn(x_ref, agg_ref, nat_ref, wux_ref, wua_ref, b1_ref, wu2_ref,
          b2_ref, out_ref):
    x_v = x_ref[...]
    agg = agg_ref[...]
    nat = nat_ref[...]
    acc = jnp.zeros((BN, H), jnp.float32) + b1_ref[...]
    for a in range(4):
      t = jnp.dot(x_v, wux_ref[a], preferred_element_type=jnp.float32)
      t = t + jnp.dot(agg, wua_ref[a], preferred_element_type=jnp.float32)
      acc = acc + nat[:, a:a + 1] * t
    u = _silu(acc)
    acc2 = jnp.zeros((BN, H), jnp.float32) + b2_ref[...]
    for a in range(4):
      acc2 = acc2 + nat[:, a:a + 1] * jnp.dot(
          u, wu2_ref[a], preferred_element_type=jnp.float32)
    out_ref[...] = x_v + acc2

  nb = NP // BN
  wspec = pl.BlockSpec((4, H, H), lambda i: (0, 0, 0))
  return pl.pallas_call(
      kfn,
      grid=(nb,),
      in_specs=[
          pl.BlockSpec((BN, H), lambda i: (i, 0)),
          pl.BlockSpec((BN, H), lambda i: (i, 0)),
          pl.BlockSpec((BN, 4), lambda i: (i, 0)),
          wspec,
          wspec,
          pl.BlockSpec((1, H), lambda i: (0, 0)),
          wspec,
          pl.BlockSpec((1, H), lambda i: (0, 0)),
      ],
      out_specs=pl.BlockSpec((BN, H), lambda i: (i, 0)),
      out_shape=jax.ShapeDtypeStruct((NP, H), jnp.float32),
  )(x, agg, nat, wux, wua, b1, wu2, b2)


def _tc_output(x, nat, nf, wo1, bo1, wo2, bo2):
  def kfn(x_ref, nat_ref, nf_ref, wo1_ref, bo1_ref, wo2_ref, bo2_ref,
          out_ref):
    x_v = x_ref[...]
    nat = nat_ref[...]
    acc = jnp.zeros((BN, H), jnp.float32) + bo1_ref[...]
    for a in range(4):
      acc = acc + nat[:, a:a + 1] * jnp.dot(
          x_v, wo1_ref[a], preferred_element_type=jnp.float32)
    u = _silu(acc)
    acc2 = jnp.zeros((BN, 128), jnp.float32) + bo2_ref[...]
    for a in range(4):
      acc2 = acc2 + nat[:, a:a + 1] * jnp.dot(
          u, wo2_ref[a], preferred_element_type=jnp.float32)
    pos = nf_ref[...][:, 0:3]
    out_ref[...] = acc2 + jnp.concatenate(
        [pos, jnp.zeros((BN, 125), jnp.float32)], axis=1)

  nb = NP // BN
  return pl.pallas_call(
      kfn,
      grid=(nb,),
      in_specs=[
          pl.BlockSpec((BN, H), lambda i: (i, 0)),
          pl.BlockSpec((BN, 4), lambda i: (i, 0)),
          pl.BlockSpec((BN, 16), lambda i: (i, 0)),
          pl.BlockSpec((4, H, H), lambda i: (0, 0, 0)),
          pl.BlockSpec((1, H), lambda i: (0, 0)),
          pl.BlockSpec((4, H, 128), lambda i: (0, 0, 0)),
          pl.BlockSpec((1, 128), lambda i: (0, 0)),
      ],
      out_specs=pl.BlockSpec((BN, 128), lambda i: (i, 0)),
      out_shape=jax.ShapeDtypeStruct((NP, 128), jnp.float32),
  )(x, nat, nf, wo1, bo1, wo2, bo2)


def _tp_weights(p):
  return p['W'].transpose(1, 0, 2), p['b'][None, :]


@jax.jit
def _run(pos, vel, charges, params, edge_index, batch):
  i32 = jnp.int32
  src = edge_index[0].astype(i32)
  dst = edge_index[1].astype(i32)
  batch = batch.astype(i32)

  nf = jnp.zeros((NP, 16), jnp.float32)
  nf = nf.at[:N, 0:3].set(pos)
  nf = nf.at[:N, 3].set(charges[:, 0])
  nf = nf.at[:N, 4].set(1.0)
  nf = nf.at[:N, 5:8].set(vel)

  pad_e = EP - E
  dst_g = jnp.concatenate([dst, jnp.zeros((pad_e,), i32)])
  src_g = jnp.concatenate([src, jnp.zeros((pad_e,), i32)])
  gidx = jnp.concatenate([dst_g, src_g]).reshape(NC, NS, K_G, CH)

  dst_p = jnp.concatenate([dst, jnp.full((pad_e,), 2 * S_NODE, i32)])
  s_lo = jnp.where(dst_p < S_NODE, dst_p, S_NODE)
  s_hi = jnp.where(dst_p >= S_NODE, dst_p - S_NODE, S_NODE)
  sidx = jnp.stack([s_lo, s_hi]).reshape(NC, NS, K_E, CH)

  pad_n = NSP - N
  bat_p = jnp.concatenate([batch, jnp.full((pad_n,), 2 * S_G, i32)])
  b_lo = jnp.where(bat_p < S_G, bat_p, S_G)
  b_hi = jnp.where(bat_p >= S_G, bat_p - S_G, S_G)
  bidx = jnp.stack([b_lo, b_hi]).reshape(NC, NS, K_B, CH)

  z16 = jnp.zeros((CH, 16), jnp.float32)
  z64 = jnp.zeros((CH, 64), jnp.float32)

  gpre = _sc_gather(nf, gidx, 16, K_G)
  ea16, escal = _tc_preproc(gpre)
  na = _sc_scatter_add(ea16, sidx, z16, 16, K_E, S_NODE, ACC_N)
  nf_sc = jnp.zeros((NSP, 16), jnp.float32).at[:NP].set(nf)
  mg = _sc_scatter_add(nf_sc, bidx, z16, 16, K_B, S_G, ACC_G)
  mp = jnp.zeros((NP, 16), jnp.float32).at[:N].set(
      jnp.repeat(mg[:G], 5, axis=0))

  w_emb, b_emb = _tp_weights(params['emb'])
  w_emb = jnp.pad(w_emb, ((0, 0), (0, 1), (0, 0)))
  x, nat = _tc_embed(nf, mp, na, w_emb, b_emb)

  for lp in params['layers']:
    w1, b1 = _tp_weights(lp['m1'])
    wxi, wxj = w1[:, :H], w1[:, H:2 * H]
    wd, wp = w1[:, 2 * H], w1[:, 2 * H + 1]
    w2, b2 = _tp_weights(lp['m2'])
    wu1, bu1 = _tp_weights(lp['u1'])
    wux, wua = wu1[:, :H], wu1[:, H:]
    wu2, bu2 = _tp_weights(lp['u2'])

    g = _sc_gather(x, gidx, H, K_G)
    m2 = _tc_edge(g, escal, wxi, wxj, wd, wp, b1, w2, b2)
    agg = _sc_scatter_add(m2, sidx, z64, H, K_E, S_NODE, ACC_N)
    x = _tc_node(x, agg, nat, wux, wua, bu1, wu2, bu2)

  wo1, bo1 = _tp_weights(params['o1'])
  wo2, bo2 = _tp_weights(params['o2'])
  wo2 = jnp.pad(wo2, ((0, 0), (0, 0), (0, 125)))
  bo2 = jnp.pad(bo2, ((0, 0), (0, 125)))
  out = _tc_output(x, nat, nf, wo1, bo1, wo2, bo2)
  return out[:N, :3]


def kernel(pos, vel, charges, params, edge_index, batch):
  return _run(pos, vel, charges, params, edge_index, batch)
